# Initial kernel scaffold; baseline (speedup 1.0000x reference)
import jax
import jax.numpy as jnp
from jax.experimental import pallas as pl


def kernel(words, chars, word_table, char_table, cconv_w, cconv_b, proj_w, proj_b, hw0_wt, hw0_bt, hw0_wg, hw0_bg, hw1_wt, hw1_bt, hw1_wg, hw1_bg):
    raise NotImplementedError("write your pallas kernel here")


# speedup vs baseline: 5.4668x; 5.4668x over previous
import functools

import numpy as np
import jax
import jax.numpy as jnp
from jax import lax
from jax.experimental import pallas as pl
from jax.experimental.pallas import tpu as pltpu
from jax.experimental.pallas import tpu_sc as plsc

B, L, WLEN = 1024, 50, 16
CV, D_CHAR, D_WORD, D_MODEL = 128, 64, 128, 128
N_TOK = B * L
W_VALID = WLEN - 4

NC, NS = 2, 16
NW = NC * NS
PER_W = N_TOK // NW
CH = 80
NCH = PER_W // CH


def _sc_word_gather(word_table, idx3):
    mesh = plsc.VectorSubcoreMesh(core_axis_name="c", subcore_axis_name="s")

    @functools.partial(
        pl.kernel,
        out_type=jax.ShapeDtypeStruct((N_TOK, D_WORD), jnp.float32),
        mesh=mesh,
        scratch_types=[
            pltpu.VMEM((NCH, CH), jnp.int32),
            pltpu.VMEM((CH, D_WORD), jnp.float32),
            pltpu.SemaphoreType.DMA,
        ],
    )
    def gather_kernel(table_hbm, idx_hbm, out_hbm, idx_v, rows_v, sem):
        wid = lax.axis_index("s") * NC + lax.axis_index("c")
        base = wid * PER_W
        pltpu.sync_copy(idx_hbm.at[wid], idx_v)

        def body(j, carry):
            pltpu.async_copy(table_hbm.at[idx_v.at[j]], rows_v, sem).wait()
            pltpu.sync_copy(rows_v, out_hbm.at[pl.ds(base + j * CH, CH)])
            return carry

        lax.fori_loop(0, NCH, body, 0)

    return gather_kernel(word_table, idx3)


BB = 8
TB = BB * L
GRID = B // BB


def _pos_tile_np():
    pos = np.arange(L)[:, None].astype(np.float64)
    i = np.arange(D_MODEL)[None, :].astype(np.float64)
    angle = pos / np.power(10000.0, (2.0 * (i // 2)) / D_MODEL)
    pe = np.where((np.arange(D_MODEL)[None, :] % 2) == 0,
                  np.sin(angle), np.cos(angle))
    return np.tile(pe.astype(np.float32), (BB, 1))


_POS_TILE = _pos_tile_np()


def _tc_body(wemb_ref, chars_ref, ctab_ref, cw_ref, cb_ref, pw_ref, pb_ref,
             wt0_ref, bt0_ref, wg0_ref, bg0_ref,
             wt1_ref, bt1_ref, wg1_ref, bg1_ref,
             pos_ref, out_ref):
    f32 = jnp.float32
    bf16 = jnp.bfloat16

    chars = chars_ref[...]
    iota = lax.broadcasted_iota(jnp.int32, (1, 1, CV), 2)
    oh = (chars[:, :, None] == iota).astype(bf16)
    oh2 = oh.reshape(TB * WLEN, CV)
    cw = cw_ref[...]
    w_cat = jnp.concatenate([cw[k] for k in range(5)], axis=1)
    t_cat = jnp.dot(ctab_ref[...], w_cat,
                    preferred_element_type=f32).astype(bf16)
    z = jnp.dot(oh2, t_cat, preferred_element_type=f32)
    z3 = z.reshape(TB, WLEN, 5 * D_CHAR)
    y = z3[:, 0:W_VALID, 0:D_CHAR]
    for k in range(1, 5):
        y = y + z3[:, k:k + W_VALID, k * D_CHAR:(k + 1) * D_CHAR]
    cemb = jnp.max(y, axis=1) + cb_ref[...]

    wemb = wemb_ref[...]
    pw = pw_ref[...]
    h = (jnp.dot(wemb.astype(bf16), pw[0:D_WORD].astype(bf16),
                 preferred_element_type=f32)
         + jnp.dot(cemb.astype(bf16), pw[D_WORD:].astype(bf16),
                   preferred_element_type=f32)
         + pb_ref[...])

    for wt_ref, bt_ref, wg_ref, bg_ref in (
            (wt0_ref, bt0_ref, wg0_ref, bg0_ref),
            (wt1_ref, bt1_ref, wg1_ref, bg1_ref)):
        hb = h.astype(bf16)
        gate = jnp.dot(hb, wg_ref[...].astype(bf16),
                       preferred_element_type=f32) + bg_ref[...]
        gate = 1.0 / (1.0 + jnp.exp(-gate))
        tr = jnp.dot(hb, wt_ref[...].astype(bf16),
                     preferred_element_type=f32) + bt_ref[...]
        tr = jnp.maximum(tr, 0.0)
        h = gate * h + (1.0 - gate) * tr

    out_ref[...] = h + pos_ref[...]


def _tc_call(wemb, chars2, char_table, cconv_w, cconv_b, proj_w2, proj_b,
             hw0_wt, hw0_bt, hw0_wg, hw0_bg, hw1_wt, hw1_bt, hw1_wg, hw1_bg,
             pos_tile):
    tok_spec = lambda w: pl.BlockSpec((TB, w), lambda i: (i, 0))
    full = lambda *shape: pl.BlockSpec(shape, lambda i: (0,) * len(shape))
    return pl.pallas_call(
        _tc_body,
        grid=(GRID,),
        in_specs=[
            tok_spec(D_WORD),
            tok_spec(WLEN),
            full(CV, D_CHAR),
            full(5, D_CHAR, D_CHAR),
            full(1, D_CHAR),
            full(D_WORD + D_CHAR, D_MODEL),
            full(1, D_MODEL),
            full(D_MODEL, D_MODEL), full(1, D_MODEL),
            full(D_MODEL, D_MODEL), full(1, D_MODEL),
            full(D_MODEL, D_MODEL), full(1, D_MODEL),
            full(D_MODEL, D_MODEL), full(1, D_MODEL),
            full(TB, D_MODEL),
        ],
        out_specs=tok_spec(D_MODEL),
        out_shape=jax.ShapeDtypeStruct((N_TOK, D_MODEL), jnp.float32),
        compiler_params=pltpu.CompilerParams(
            dimension_semantics=("parallel",)),
    )(wemb, chars2, char_table, cconv_w, cconv_b.reshape(1, D_CHAR),
      proj_w2, proj_b.reshape(1, D_MODEL),
      hw0_wt, hw0_bt.reshape(1, D_MODEL), hw0_wg, hw0_bg.reshape(1, D_MODEL),
      hw1_wt, hw1_bt.reshape(1, D_MODEL), hw1_wg, hw1_bg.reshape(1, D_MODEL),
      pos_tile)


def kernel(words, chars, word_table, char_table, cconv_w, cconv_b, proj_w,
           proj_b, hw0_wt, hw0_bt, hw0_wg, hw0_bg,
           hw1_wt, hw1_bt, hw1_wg, hw1_bg):
    idx3 = words.reshape(NW, NCH, CH)
    wemb = _sc_word_gather(word_table, idx3)
    chars2 = chars.reshape(N_TOK, WLEN)
    pos_tile = jnp.asarray(_POS_TILE)
    out2 = _tc_call(wemb, chars2, char_table, cconv_w, cconv_b,
                    proj_w.reshape(D_WORD + D_CHAR, D_MODEL), proj_b,
                    hw0_wt, hw0_bt, hw0_wg, hw0_bg,
                    hw1_wt, hw1_bt, hw1_wg, hw1_bg, pos_tile)
    return out2.reshape(B, L, D_MODEL)


# speedup vs baseline: 5.6617x; 1.0357x over previous
import functools

import numpy as np
import jax
import jax.numpy as jnp
from jax import lax
from jax.experimental import pallas as pl
from jax.experimental.pallas import tpu as pltpu
from jax.experimental.pallas import tpu_sc as plsc

B, L, WLEN = 1024, 50, 16
CV, D_CHAR, D_WORD, D_MODEL = 128, 64, 128, 128
N_TOK = B * L
W_VALID = WLEN - 4

NC, NS = 2, 16
NW = NC * NS
PER_W = N_TOK // NW
CH = 80
NCH = PER_W // CH


def _sc_word_gather(word_table, idx3):
    mesh = plsc.VectorSubcoreMesh(core_axis_name="c", subcore_axis_name="s")

    @functools.partial(
        pl.kernel,
        out_type=jax.ShapeDtypeStruct((N_TOK, D_WORD), jnp.float32),
        mesh=mesh,
        scratch_types=[
            pltpu.VMEM((NCH, CH), jnp.int32),
            pltpu.VMEM((CH, D_WORD), jnp.float32),
            pltpu.SemaphoreType.DMA,
        ],
    )
    def gather_kernel(table_hbm, idx_hbm, out_hbm, idx_v, rows_v, sem):
        wid = lax.axis_index("s") * NC + lax.axis_index("c")
        base = wid * PER_W
        pltpu.sync_copy(idx_hbm.at[wid], idx_v)

        def body(j, carry):
            pltpu.async_copy(table_hbm.at[idx_v.at[j]], rows_v, sem).wait()
            pltpu.sync_copy(rows_v, out_hbm.at[pl.ds(base + j * CH, CH)])
            return carry

        lax.fori_loop(0, NCH, body, 0)

    return gather_kernel(word_table, idx3)


TB = 256
GRID = N_TOK // TB


def _pos_np():
    pos = np.arange(L)[:, None].astype(np.float64)
    i = np.arange(D_MODEL)[None, :].astype(np.float64)
    angle = pos / np.power(10000.0, (2.0 * (i // 2)) / D_MODEL)
    pe = np.where((np.arange(D_MODEL)[None, :] % 2) == 0,
                  np.sin(angle), np.cos(angle))
    return pe.astype(np.float32)


_POS = _pos_np()


def _tc_body(wemb_ref, chars_ref, ctab_ref, cw_ref, cb_ref, pw_ref, pb_ref,
             wt0_ref, bt0_ref, wg0_ref, bg0_ref,
             wt1_ref, bt1_ref, wg1_ref, bg1_ref,
             pos_ref, out_ref):
    f32 = jnp.float32
    bf16 = jnp.bfloat16

    chars = chars_ref[...]
    iota = lax.broadcasted_iota(jnp.int32, (1, 1, CV), 2)
    oh = (chars[:, :, None] == iota).astype(bf16)
    oh2 = oh.reshape(TB * WLEN, CV)
    cw = cw_ref[...]
    w_cat = jnp.concatenate([cw[k] for k in range(5)], axis=1)
    t_cat = jnp.dot(ctab_ref[...], w_cat,
                    preferred_element_type=f32).astype(bf16)
    z = jnp.dot(oh2, t_cat, preferred_element_type=f32)
    z3 = z.reshape(TB, WLEN, 5 * D_CHAR)
    y = z3[:, 0:W_VALID, 0:D_CHAR]
    for k in range(1, 5):
        y = y + z3[:, k:k + W_VALID, k * D_CHAR:(k + 1) * D_CHAR]
    cemb = jnp.max(y, axis=1) + cb_ref[...]

    wemb = wemb_ref[...]
    pw = pw_ref[...]
    h = (jnp.dot(wemb.astype(bf16), pw[0:D_WORD].astype(bf16),
                 preferred_element_type=f32)
         + jnp.dot(cemb.astype(bf16), pw[D_WORD:].astype(bf16),
                   preferred_element_type=f32)
         + pb_ref[...])

    for wt_ref, bt_ref, wg_ref, bg_ref in (
            (wt0_ref, bt0_ref, wg0_ref, bg0_ref),
            (wt1_ref, bt1_ref, wg1_ref, bg1_ref)):
        hb = h.astype(bf16)
        gate = jnp.dot(hb, wg_ref[...].astype(bf16),
                       preferred_element_type=f32) + bg_ref[...]
        gate = 1.0 / (1.0 + jnp.exp(-gate))
        tr = jnp.dot(hb, wt_ref[...].astype(bf16),
                     preferred_element_type=f32) + bt_ref[...]
        tr = jnp.maximum(tr, 0.0)
        h = gate * h + (1.0 - gate) * tr

    out_ref[...] = h + pos_ref[0]


def _tc_call(wemb, chars2, char_table, cconv_w, cconv_b, proj_w2, proj_b,
             hw0_wt, hw0_bt, hw0_wg, hw0_bg, hw1_wt, hw1_bt, hw1_wg, hw1_bg,
             pos_tile):
    tok_spec = lambda w: pl.BlockSpec((TB, w), lambda i: (i, 0))
    full = lambda *shape: pl.BlockSpec(shape, lambda i: (0,) * len(shape))
    return pl.pallas_call(
        _tc_body,
        grid=(GRID,),
        in_specs=[
            tok_spec(D_WORD),
            tok_spec(WLEN),
            full(CV, D_CHAR),
            full(5, D_CHAR, D_CHAR),
            full(1, D_CHAR),
            full(D_WORD + D_CHAR, D_MODEL),
            full(1, D_MODEL),
            full(D_MODEL, D_MODEL), full(1, D_MODEL),
            full(D_MODEL, D_MODEL), full(1, D_MODEL),
            full(D_MODEL, D_MODEL), full(1, D_MODEL),
            full(D_MODEL, D_MODEL), full(1, D_MODEL),
            pl.BlockSpec((1, 1, D_MODEL), lambda i: (i // (B // TB), 0, 0)),
        ],
        out_specs=tok_spec(D_MODEL),
        out_shape=jax.ShapeDtypeStruct((N_TOK, D_MODEL), jnp.float32),
        compiler_params=pltpu.CompilerParams(
            dimension_semantics=("parallel",)),
    )(wemb, chars2, char_table, cconv_w, cconv_b.reshape(1, D_CHAR),
      proj_w2, proj_b.reshape(1, D_MODEL),
      hw0_wt, hw0_bt.reshape(1, D_MODEL), hw0_wg, hw0_bg.reshape(1, D_MODEL),
      hw1_wt, hw1_bt.reshape(1, D_MODEL), hw1_wg, hw1_bg.reshape(1, D_MODEL),
      pos_tile)


def kernel(words, chars, word_table, char_table, cconv_w, cconv_b, proj_w,
           proj_b, hw0_wt, hw0_bt, hw0_wg, hw0_bg,
           hw1_wt, hw1_bt, hw1_wg, hw1_bg):
    idx3 = words.T.reshape(NW, NCH, CH)
    wemb = _sc_word_gather(word_table, idx3)
    chars2 = chars.transpose(1, 0, 2).reshape(N_TOK, WLEN)
    out2 = _tc_call(wemb, chars2, char_table, cconv_w, cconv_b,
                    proj_w.reshape(D_WORD + D_CHAR, D_MODEL), proj_b,
                    hw0_wt, hw0_bt, hw0_wg, hw0_bg,
                    hw1_wt, hw1_bt, hw1_wg, hw1_bg,
                    jnp.asarray(_POS).reshape(L, 1, D_MODEL))
    return out2.reshape(L, B, D_MODEL).transpose(1, 0, 2)


# speedup vs baseline: 5.6645x; 1.0005x over previous
import functools

import numpy as np
import jax
import jax.numpy as jnp
from jax import lax
from jax.experimental import pallas as pl
from jax.experimental.pallas import tpu as pltpu
from jax.experimental.pallas import tpu_sc as plsc

B, L, WLEN = 1024, 50, 16
CV, D_CHAR, D_WORD, D_MODEL = 128, 64, 128, 128
N_TOK = B * L
W_VALID = WLEN - 4

NC, NS = 2, 16
NW = NC * NS
PER_W = N_TOK // NW
CH = 80
NCH = PER_W // CH


def _sc_word_gather(word_table, idx3):
    mesh = plsc.VectorSubcoreMesh(core_axis_name="c", subcore_axis_name="s")

    @functools.partial(
        pl.kernel,
        out_type=jax.ShapeDtypeStruct((N_TOK, D_WORD), jnp.float32),
        mesh=mesh,
        scratch_types=[
            pltpu.VMEM((NCH, CH), jnp.int32),
            pltpu.VMEM((CH, D_WORD), jnp.float32),
            pltpu.SemaphoreType.DMA,
        ],
    )
    def gather_kernel(table_hbm, idx_hbm, out_hbm, idx_v, rows_v, sem):
        wid = lax.axis_index("s") * NC + lax.axis_index("c")
        base = wid * PER_W
        pltpu.sync_copy(idx_hbm.at[wid], idx_v)

        def body(j, carry):
            pltpu.async_copy(table_hbm.at[idx_v.at[j]], rows_v, sem).wait()
            pltpu.sync_copy(rows_v, out_hbm.at[pl.ds(base + j * CH, CH)])
            return carry

        lax.fori_loop(0, NCH, body, 0)

    return gather_kernel(word_table, idx3)


TB = 512
GRID = N_TOK // TB


def _pos_np():
    pos = np.arange(L)[:, None].astype(np.float64)
    i = np.arange(D_MODEL)[None, :].astype(np.float64)
    angle = pos / np.power(10000.0, (2.0 * (i // 2)) / D_MODEL)
    pe = np.where((np.arange(D_MODEL)[None, :] % 2) == 0,
                  np.sin(angle), np.cos(angle))
    return pe.astype(np.float32)


_POS = _pos_np()


def _tc_body(wemb_ref, chars_ref, ctab_ref, cw_ref, cb_ref, pw_ref, pb_ref,
             wt0_ref, bt0_ref, wg0_ref, bg0_ref,
             wt1_ref, bt1_ref, wg1_ref, bg1_ref,
             pos_ref, out_ref):
    f32 = jnp.float32
    bf16 = jnp.bfloat16

    chars = chars_ref[...]
    iota = lax.broadcasted_iota(jnp.int32, (1, 1, CV), 2)
    oh = (chars[:, :, None] == iota).astype(bf16)
    oh2 = oh.reshape(TB * WLEN, CV)
    cw = cw_ref[...]
    w_cat = jnp.concatenate([cw[k] for k in range(5)], axis=1)
    t_cat = jnp.dot(ctab_ref[...], w_cat,
                    preferred_element_type=f32).astype(bf16)
    z = jnp.dot(oh2, t_cat, preferred_element_type=f32)
    z3 = z.reshape(TB, WLEN, 5 * D_CHAR)
    y = z3[:, 0:W_VALID, 0:D_CHAR]
    for k in range(1, 5):
        y = y + z3[:, k:k + W_VALID, k * D_CHAR:(k + 1) * D_CHAR]
    cemb = jnp.max(y, axis=1) + cb_ref[...]

    wemb = wemb_ref[...]
    pw = pw_ref[...]
    h = (jnp.dot(wemb.astype(bf16), pw[0:D_WORD].astype(bf16),
                 preferred_element_type=f32)
         + jnp.dot(cemb.astype(bf16), pw[D_WORD:].astype(bf16),
                   preferred_element_type=f32)
         + pb_ref[...])

    for wt_ref, bt_ref, wg_ref, bg_ref in (
            (wt0_ref, bt0_ref, wg0_ref, bg0_ref),
            (wt1_ref, bt1_ref, wg1_ref, bg1_ref)):
        hb = h.astype(bf16)
        gate = jnp.dot(hb, wg_ref[...].astype(bf16),
                       preferred_element_type=f32) + bg_ref[...]
        gate = 1.0 / (1.0 + jnp.exp(-gate))
        tr = jnp.dot(hb, wt_ref[...].astype(bf16),
                     preferred_element_type=f32) + bt_ref[...]
        tr = jnp.maximum(tr, 0.0)
        h = gate * h + (1.0 - gate) * tr

    out_ref[...] = h + pos_ref[0]


def _tc_call(wemb, chars2, char_table, cconv_w, cconv_b, proj_w2, proj_b,
             hw0_wt, hw0_bt, hw0_wg, hw0_bg, hw1_wt, hw1_bt, hw1_wg, hw1_bg,
             pos_tile):
    tok_spec = lambda w: pl.BlockSpec((TB, w), lambda i: (i, 0))
    full = lambda *shape: pl.BlockSpec(shape, lambda i: (0,) * len(shape))
    return pl.pallas_call(
        _tc_body,
        grid=(GRID,),
        in_specs=[
            tok_spec(D_WORD),
            tok_spec(WLEN),
            full(CV, D_CHAR),
            full(5, D_CHAR, D_CHAR),
            full(1, D_CHAR),
            full(D_WORD + D_CHAR, D_MODEL),
            full(1, D_MODEL),
            full(D_MODEL, D_MODEL), full(1, D_MODEL),
            full(D_MODEL, D_MODEL), full(1, D_MODEL),
            full(D_MODEL, D_MODEL), full(1, D_MODEL),
            full(D_MODEL, D_MODEL), full(1, D_MODEL),
            pl.BlockSpec((1, 1, D_MODEL), lambda i: (i // (B // TB), 0, 0)),
        ],
        out_specs=tok_spec(D_MODEL),
        out_shape=jax.ShapeDtypeStruct((N_TOK, D_MODEL), jnp.float32),
        compiler_params=pltpu.CompilerParams(
            dimension_semantics=("parallel",)),
    )(wemb, chars2, char_table, cconv_w, cconv_b.reshape(1, D_CHAR),
      proj_w2, proj_b.reshape(1, D_MODEL),
      hw0_wt, hw0_bt.reshape(1, D_MODEL), hw0_wg, hw0_bg.reshape(1, D_MODEL),
      hw1_wt, hw1_bt.reshape(1, D_MODEL), hw1_wg, hw1_bg.reshape(1, D_MODEL),
      pos_tile)


def kernel(words, chars, word_table, char_table, cconv_w, cconv_b, proj_w,
           proj_b, hw0_wt, hw0_bt, hw0_wg, hw0_bg,
           hw1_wt, hw1_bt, hw1_wg, hw1_bg):
    idx3 = words.T.reshape(NW, NCH, CH)
    wemb = _sc_word_gather(word_table, idx3)
    chars2 = chars.transpose(1, 0, 2).reshape(N_TOK, WLEN)
    out2 = _tc_call(wemb, chars2, char_table, cconv_w, cconv_b,
                    proj_w.reshape(D_WORD + D_CHAR, D_MODEL), proj_b,
                    hw0_wt, hw0_bt, hw0_wg, hw0_bg,
                    hw1_wt, hw1_bt, hw1_wg, hw1_bg,
                    jnp.asarray(_POS).reshape(L, 1, D_MODEL))
    return out2.reshape(L, B, D_MODEL).transpose(1, 0, 2)


# speedup vs baseline: 10.5855x; 1.8687x over previous
import functools

import numpy as np
import jax
import jax.numpy as jnp
from jax import lax
from jax.experimental import pallas as pl
from jax.experimental.pallas import tpu as pltpu
from jax.experimental.pallas import tpu_sc as plsc

B, L, WLEN = 1024, 50, 16
CV, D_CHAR, D_WORD, D_MODEL = 128, 64, 128, 128
N_TOK = B * L
W_VALID = WLEN - 4

NC, NS = 2, 16
NW = NC * NS
PER_W = N_TOK // NW
CH = 80
NCH = PER_W // CH


def _sc_word_gather(word_table, idx3):
    mesh = plsc.VectorSubcoreMesh(core_axis_name="c", subcore_axis_name="s")

    @functools.partial(
        pl.kernel,
        out_type=jax.ShapeDtypeStruct((N_TOK, D_WORD), jnp.float32),
        mesh=mesh,
        scratch_types=[
            pltpu.VMEM((NCH, CH), jnp.int32),
            pltpu.VMEM((CH, D_WORD), jnp.float32),
            pltpu.SemaphoreType.DMA,
        ],
    )
    def gather_kernel(table_hbm, idx_hbm, out_hbm, idx_v, rows_v, sem):
        wid = lax.axis_index("s") * NC + lax.axis_index("c")
        base = wid * PER_W
        pltpu.sync_copy(idx_hbm.at[wid], idx_v)

        def body(j, carry):
            pltpu.async_copy(table_hbm.at[idx_v.at[j]], rows_v, sem).wait()
            pltpu.sync_copy(rows_v, out_hbm.at[pl.ds(base + j * CH, CH)])
            return carry

        lax.fori_loop(0, NCH, body, 0)

    return gather_kernel(word_table, idx3)


TB = 512
GRID = N_TOK // TB


def _pos_np():
    pos = np.arange(L)[:, None].astype(np.float64)
    i = np.arange(D_MODEL)[None, :].astype(np.float64)
    angle = pos / np.power(10000.0, (2.0 * (i // 2)) / D_MODEL)
    pe = np.where((np.arange(D_MODEL)[None, :] % 2) == 0,
                  np.sin(angle), np.cos(angle))
    return pe.astype(np.float32)


_POS = _pos_np()


def _tc_body(wemb_ref, chars_ref, ctab_ref, cw_ref, cb_ref, pw_ref, pb_ref,
             wt0_ref, bt0_ref, wg0_ref, bg0_ref,
             wt1_ref, bt1_ref, wg1_ref, bg1_ref,
             pos_ref, out_ref):
    f32 = jnp.float32
    bf16 = jnp.bfloat16

    chars = chars_ref[...]
    cflat = chars.reshape(1, WLEN * TB)
    iota = lax.broadcasted_iota(jnp.int32, (CV, 1), 0)
    oht = (cflat == iota).astype(bf16)
    cw = cw_ref[...]
    ctab = ctab_ref[...]
    t_cat_t = jnp.concatenate(
        [lax.dot_general(cw[k], ctab, (((0,), (1,)), ((), ())),
                         preferred_element_type=f32)
         for k in range(5)], axis=0).astype(bf16)
    zt = jnp.dot(t_cat_t, oht, preferred_element_type=f32)
    yt = zt[0:D_CHAR, 0:W_VALID * TB]
    for k in range(1, 5):
        yt = yt + zt[k * D_CHAR:(k + 1) * D_CHAR, k * TB:(k + W_VALID) * TB]
    cembt = yt[:, 0:TB]
    for p in range(1, W_VALID):
        cembt = jnp.maximum(cembt, yt[:, p * TB:(p + 1) * TB])
    cemb = cembt.T + cb_ref[...]

    wemb = wemb_ref[...]
    pw = pw_ref[...]
    h = (jnp.dot(wemb.astype(bf16), pw[0:D_WORD].astype(bf16),
                 preferred_element_type=f32)
         + jnp.dot(cemb.astype(bf16), pw[D_WORD:].astype(bf16),
                   preferred_element_type=f32)
         + pb_ref[...])

    for wt_ref, bt_ref, wg_ref, bg_ref in (
            (wt0_ref, bt0_ref, wg0_ref, bg0_ref),
            (wt1_ref, bt1_ref, wg1_ref, bg1_ref)):
        hb = h.astype(bf16)
        gate = jnp.dot(hb, wg_ref[...].astype(bf16),
                       preferred_element_type=f32) + bg_ref[...]
        gate = 1.0 / (1.0 + jnp.exp(-gate))
        tr = jnp.dot(hb, wt_ref[...].astype(bf16),
                     preferred_element_type=f32) + bt_ref[...]
        tr = jnp.maximum(tr, 0.0)
        h = gate * h + (1.0 - gate) * tr

    out_ref[...] = h + pos_ref[0]


def _tc_call(wemb, chars2, char_table, cconv_w, cconv_b, proj_w2, proj_b,
             hw0_wt, hw0_bt, hw0_wg, hw0_bg, hw1_wt, hw1_bt, hw1_wg, hw1_bg,
             pos_tile):
    tok_spec = lambda w: pl.BlockSpec((TB, w), lambda i: (i, 0))
    full = lambda *shape: pl.BlockSpec(shape, lambda i: (0,) * len(shape))
    return pl.pallas_call(
        _tc_body,
        grid=(GRID,),
        in_specs=[
            tok_spec(D_WORD),
            pl.BlockSpec((WLEN, TB), lambda i: (0, i)),
            full(CV, D_CHAR),
            full(5, D_CHAR, D_CHAR),
            full(1, D_CHAR),
            full(D_WORD + D_CHAR, D_MODEL),
            full(1, D_MODEL),
            full(D_MODEL, D_MODEL), full(1, D_MODEL),
            full(D_MODEL, D_MODEL), full(1, D_MODEL),
            full(D_MODEL, D_MODEL), full(1, D_MODEL),
            full(D_MODEL, D_MODEL), full(1, D_MODEL),
            pl.BlockSpec((1, 1, D_MODEL), lambda i: (i // (B // TB), 0, 0)),
        ],
        out_specs=tok_spec(D_MODEL),
        out_shape=jax.ShapeDtypeStruct((N_TOK, D_MODEL), jnp.float32),
        compiler_params=pltpu.CompilerParams(
            dimension_semantics=("parallel",)),
    )(wemb, chars2, char_table, cconv_w, cconv_b.reshape(1, D_CHAR),
      proj_w2, proj_b.reshape(1, D_MODEL),
      hw0_wt, hw0_bt.reshape(1, D_MODEL), hw0_wg, hw0_bg.reshape(1, D_MODEL),
      hw1_wt, hw1_bt.reshape(1, D_MODEL), hw1_wg, hw1_bg.reshape(1, D_MODEL),
      pos_tile)


def kernel(words, chars, word_table, char_table, cconv_w, cconv_b, proj_w,
           proj_b, hw0_wt, hw0_bt, hw0_wg, hw0_bg,
           hw1_wt, hw1_bt, hw1_wg, hw1_bg):
    idx3 = words.T.reshape(NW, NCH, CH)
    wemb = _sc_word_gather(word_table, idx3)
    chars2 = chars.transpose(2, 1, 0).reshape(WLEN, N_TOK)
    out2 = _tc_call(wemb, chars2, char_table, cconv_w, cconv_b,
                    proj_w.reshape(D_WORD + D_CHAR, D_MODEL), proj_b,
                    hw0_wt, hw0_bt, hw0_wg, hw0_bg,
                    hw1_wt, hw1_bt, hw1_wg, hw1_bg,
                    jnp.asarray(_POS).reshape(L, 1, D_MODEL))
    return out2.reshape(L, B, D_MODEL).transpose(1, 0, 2)


# speedup vs baseline: 10.7494x; 1.0155x over previous
import functools

import numpy as np
import jax
import jax.numpy as jnp
from jax import lax
from jax.experimental import pallas as pl
from jax.experimental.pallas import tpu as pltpu
from jax.experimental.pallas import tpu_sc as plsc

B, L, WLEN = 1024, 50, 16
CV, D_CHAR, D_WORD, D_MODEL = 128, 64, 128, 128
N_TOK = B * L
W_VALID = WLEN - 4

NC, NS = 2, 16
NW = NC * NS
PER_W = N_TOK // NW
CH = 80
NCH = PER_W // CH


def _sc_word_gather(word_table, idx3):
    mesh = plsc.VectorSubcoreMesh(core_axis_name="c", subcore_axis_name="s")

    @functools.partial(
        pl.kernel,
        out_type=jax.ShapeDtypeStruct((N_TOK, D_WORD), jnp.float32),
        mesh=mesh,
        scratch_types=[
            pltpu.VMEM((NCH, CH), jnp.int32),
            pltpu.VMEM((CH, D_WORD), jnp.float32),
            pltpu.SemaphoreType.DMA,
        ],
    )
    def gather_kernel(table_hbm, idx_hbm, out_hbm, idx_v, rows_v, sem):
        wid = lax.axis_index("s") * NC + lax.axis_index("c")
        base = wid * PER_W
        pltpu.sync_copy(idx_hbm.at[wid], idx_v)

        def body(j, carry):
            pltpu.async_copy(table_hbm.at[idx_v.at[j]], rows_v, sem).wait()
            pltpu.sync_copy(rows_v, out_hbm.at[pl.ds(base + j * CH, CH)])
            return carry

        lax.fori_loop(0, NCH, body, 0)

    return gather_kernel(word_table, idx3)


TB = 512
GRID = N_TOK // TB


def _pos_np():
    pos = np.arange(L)[:, None].astype(np.float64)
    i = np.arange(D_MODEL)[None, :].astype(np.float64)
    angle = pos / np.power(10000.0, (2.0 * (i // 2)) / D_MODEL)
    pe = np.where((np.arange(D_MODEL)[None, :] % 2) == 0,
                  np.sin(angle), np.cos(angle))
    return pe.astype(np.float32)


_POS = _pos_np()


def _tc_char_body(chars_ref, ctab_ref, cw_ref, cb_ref, cemb_ref):
    f32 = jnp.float32
    bf16 = jnp.bfloat16
    chars = chars_ref[...]
    cflat = chars.reshape(1, WLEN * TB)
    iota = lax.broadcasted_iota(jnp.int32, (CV, 1), 0)
    oht = (cflat == iota).astype(bf16)
    cw = cw_ref[...]
    ctab = ctab_ref[...]
    t_cat_t = jnp.concatenate(
        [lax.dot_general(cw[k], ctab, (((0,), (1,)), ((), ())),
                         preferred_element_type=f32)
         for k in range(5)], axis=0).astype(bf16)
    zt = jnp.dot(t_cat_t, oht, preferred_element_type=f32)
    yt = zt[0:D_CHAR, 0:W_VALID * TB]
    for k in range(1, 5):
        yt = yt + zt[k * D_CHAR:(k + 1) * D_CHAR, k * TB:(k + W_VALID) * TB]
    cembt = yt[:, 0:TB]
    for p in range(1, W_VALID):
        cembt = jnp.maximum(cembt, yt[:, p * TB:(p + 1) * TB])
    cemb_ref[...] = cembt.T + cb_ref[...]


def _tc_mix_body(wemb_ref, cemb_ref, pw_ref, pb_ref,
                 wt0_ref, bt0_ref, wg0_ref, bg0_ref,
                 wt1_ref, bt1_ref, wg1_ref, bg1_ref,
                 pos_ref, out_ref):
    f32 = jnp.float32
    bf16 = jnp.bfloat16
    wemb = wemb_ref[...]
    pw = pw_ref[...]
    h = (jnp.dot(wemb.astype(bf16), pw[0:D_WORD].astype(bf16),
                 preferred_element_type=f32)
         + jnp.dot(cemb_ref[...].astype(bf16), pw[D_WORD:].astype(bf16),
                   preferred_element_type=f32)
         + pb_ref[...])
    for wt_ref, bt_ref, wg_ref, bg_ref in (
            (wt0_ref, bt0_ref, wg0_ref, bg0_ref),
            (wt1_ref, bt1_ref, wg1_ref, bg1_ref)):
        hb = h.astype(bf16)
        gate = jnp.dot(hb, wg_ref[...].astype(bf16),
                       preferred_element_type=f32) + bg_ref[...]
        gate = 1.0 / (1.0 + jnp.exp(-gate))
        tr = jnp.dot(hb, wt_ref[...].astype(bf16),
                     preferred_element_type=f32) + bt_ref[...]
        tr = jnp.maximum(tr, 0.0)
        h = gate * h + (1.0 - gate) * tr
    out_ref[...] = h + pos_ref[0]


def _tc_char_call(chars2, char_table, cconv_w, cconv_b):
    full = lambda *shape: pl.BlockSpec(shape, lambda i: (0,) * len(shape))
    return pl.pallas_call(
        _tc_char_body,
        grid=(GRID,),
        in_specs=[
            pl.BlockSpec((WLEN, TB), lambda i: (0, i)),
            full(CV, D_CHAR),
            full(5, D_CHAR, D_CHAR),
            full(1, D_CHAR),
        ],
        out_specs=pl.BlockSpec((TB, D_CHAR), lambda i: (i, 0)),
        out_shape=jax.ShapeDtypeStruct((N_TOK, D_CHAR), jnp.float32),
        compiler_params=pltpu.CompilerParams(
            dimension_semantics=("parallel",)),
    )(chars2, char_table, cconv_w, cconv_b.reshape(1, D_CHAR))


def _tc_mix_call(wemb, cemb2, proj_w2, proj_b,
                 hw0_wt, hw0_bt, hw0_wg, hw0_bg, hw1_wt, hw1_bt, hw1_wg, hw1_bg):
    tok_spec = lambda w: pl.BlockSpec((TB, w), lambda i: (i, 0))
    full = lambda *shape: pl.BlockSpec(shape, lambda i: (0,) * len(shape))
    return pl.pallas_call(
        _tc_mix_body,
        grid=(GRID,),
        in_specs=[
            tok_spec(D_WORD),
            tok_spec(D_CHAR),
            full(D_WORD + D_CHAR, D_MODEL),
            full(1, D_MODEL),
            full(D_MODEL, D_MODEL), full(1, D_MODEL),
            full(D_MODEL, D_MODEL), full(1, D_MODEL),
            full(D_MODEL, D_MODEL), full(1, D_MODEL),
            full(D_MODEL, D_MODEL), full(1, D_MODEL),
            pl.BlockSpec((1, 1, D_MODEL), lambda i: (i // (B // TB), 0, 0)),
        ],
        out_specs=tok_spec(D_MODEL),
        out_shape=jax.ShapeDtypeStruct((N_TOK, D_MODEL), jnp.float32),
        compiler_params=pltpu.CompilerParams(
            dimension_semantics=("parallel",)),
    )(wemb, cemb2, proj_w2, proj_b.reshape(1, D_MODEL),
      hw0_wt, hw0_bt.reshape(1, D_MODEL), hw0_wg, hw0_bg.reshape(1, D_MODEL),
      hw1_wt, hw1_bt.reshape(1, D_MODEL), hw1_wg, hw1_bg.reshape(1, D_MODEL),
      jnp.asarray(_POS).reshape(L, 1, D_MODEL))


def kernel(words, chars, word_table, char_table, cconv_w, cconv_b, proj_w,
           proj_b, hw0_wt, hw0_bt, hw0_wg, hw0_bg,
           hw1_wt, hw1_bt, hw1_wg, hw1_bg):
    idx3 = words.T.reshape(NW, NCH, CH)
    wemb = _sc_word_gather(word_table, idx3)
    chars2 = chars.transpose(2, 1, 0).reshape(WLEN, N_TOK)
    cemb2 = _tc_char_call(chars2, char_table, cconv_w, cconv_b)
    out2 = _tc_mix_call(wemb, cemb2,
                        proj_w.reshape(D_WORD + D_CHAR, D_MODEL), proj_b,
                        hw0_wt, hw0_bt, hw0_wg, hw0_bg,
                        hw1_wt, hw1_bt, hw1_wg, hw1_bg)
    return out2.reshape(L, B, D_MODEL).transpose(1, 0, 2)


# speedup vs baseline: 14.1427x; 1.3157x over previous
import functools

import numpy as np
import jax
import jax.numpy as jnp
from jax import lax
from jax.experimental import pallas as pl
from jax.experimental.pallas import tpu as pltpu
from jax.experimental.pallas import tpu_sc as plsc

B, L, WLEN = 1024, 50, 16
CV, D_CHAR, D_WORD, D_MODEL = 128, 64, 128, 128
N_TOK = B * L
W_VALID = WLEN - 4

NC, NS = 2, 16
NW = NC * NS
PER_W = N_TOK // NW
CH = 80
NCH = PER_W // CH


def _sc_word_gather(word_table, idx3):
    mesh = plsc.VectorSubcoreMesh(core_axis_name="c", subcore_axis_name="s")

    @functools.partial(
        pl.kernel,
        out_type=jax.ShapeDtypeStruct((N_TOK, D_WORD), jnp.float32),
        mesh=mesh,
        scratch_types=[
            pltpu.VMEM((NCH, CH), jnp.int32),
            pltpu.VMEM((CH, D_WORD), jnp.float32),
            pltpu.SemaphoreType.DMA,
        ],
    )
    def gather_kernel(table_hbm, idx_hbm, out_hbm, idx_v, rows_v, sem):
        wid = lax.axis_index("s") * NC + lax.axis_index("c")
        base = wid * PER_W
        pltpu.sync_copy(idx_hbm.at[wid], idx_v)

        def body(j, carry):
            pltpu.async_copy(table_hbm.at[idx_v.at[j]], rows_v, sem).wait()
            pltpu.sync_copy(rows_v, out_hbm.at[pl.ds(base + j * CH, CH)])
            return carry

        lax.fori_loop(0, NCH, body, 0)

    return gather_kernel(word_table, idx3)


TB = 1024
GRID = N_TOK // TB
TBM = 2048
GRIDM = N_TOK // TBM
LPB = TBM // B


def _pos_np():
    pos = np.arange(L)[:, None].astype(np.float64)
    i = np.arange(D_MODEL)[None, :].astype(np.float64)
    angle = pos / np.power(10000.0, (2.0 * (i // 2)) / D_MODEL)
    pe = np.where((np.arange(D_MODEL)[None, :] % 2) == 0,
                  np.sin(angle), np.cos(angle))
    return pe.astype(np.float32)


_POS = _pos_np()


def _tc_char_body(chars_ref, ctab_ref, cw_ref, cb_ref, cemb_ref):
    f32 = jnp.float32
    bf16 = jnp.bfloat16
    chars = chars_ref[...]
    cflat = chars.reshape(1, WLEN * TB)
    iota = lax.broadcasted_iota(jnp.int32, (CV, 1), 0)
    oht = (cflat == iota).astype(bf16)
    cw = cw_ref[...]
    ctab = ctab_ref[...]
    t_cat_t = jnp.concatenate(
        [lax.dot_general(cw[k], ctab, (((0,), (1,)), ((), ())),
                         preferred_element_type=f32)
         for k in range(5)], axis=0).astype(bf16)
    zt = jnp.dot(t_cat_t, oht, preferred_element_type=f32)
    yt = zt[0:D_CHAR, 0:W_VALID * TB]
    for k in range(1, 5):
        yt = yt + zt[k * D_CHAR:(k + 1) * D_CHAR, k * TB:(k + W_VALID) * TB]
    cembt = yt[:, 0:TB]
    for p in range(1, W_VALID):
        cembt = jnp.maximum(cembt, yt[:, p * TB:(p + 1) * TB])
    cemb_ref[...] = cembt.T + cb_ref[...]


def _tc_mix_body(wemb_ref, cemb_ref, pw_ref, pb_ref,
                 wt0_ref, bt0_ref, wg0_ref, bg0_ref,
                 wt1_ref, bt1_ref, wg1_ref, bg1_ref,
                 pos_ref, out_ref):
    f32 = jnp.float32
    bf16 = jnp.bfloat16
    wemb = wemb_ref[...]
    pw = pw_ref[...]
    h = (jnp.dot(wemb.astype(bf16), pw[0:D_WORD].astype(bf16),
                 preferred_element_type=f32)
         + jnp.dot(cemb_ref[...].astype(bf16), pw[D_WORD:].astype(bf16),
                   preferred_element_type=f32)
         + pb_ref[...])
    for wt_ref, bt_ref, wg_ref, bg_ref in (
            (wt0_ref, bt0_ref, wg0_ref, bg0_ref),
            (wt1_ref, bt1_ref, wg1_ref, bg1_ref)):
        hb = h.astype(bf16)
        gate = jnp.dot(hb, wg_ref[...].astype(bf16),
                       preferred_element_type=f32) + bg_ref[...]
        gate = 1.0 / (1.0 + jnp.exp(-gate))
        tr = jnp.dot(hb, wt_ref[...].astype(bf16),
                     preferred_element_type=f32) + bt_ref[...]
        tr = jnp.maximum(tr, 0.0)
        h = gate * h + (1.0 - gate) * tr
    pos = jnp.broadcast_to(pos_ref[:, 0, :][:, None, :],
                           (LPB, B, D_MODEL)).reshape(TBM, D_MODEL)
    out_ref[...] = h + pos


def _tc_char_call(chars2, char_table, cconv_w, cconv_b):
    full = lambda *shape: pl.BlockSpec(shape, lambda i: (0,) * len(shape))
    return pl.pallas_call(
        _tc_char_body,
        grid=(GRID,),
        in_specs=[
            pl.BlockSpec((WLEN, TB), lambda i: (0, i)),
            full(CV, D_CHAR),
            full(5, D_CHAR, D_CHAR),
            full(1, D_CHAR),
        ],
        out_specs=pl.BlockSpec((TB, D_CHAR), lambda i: (i, 0)),
        out_shape=jax.ShapeDtypeStruct((N_TOK, D_CHAR), jnp.float32),
        compiler_params=pltpu.CompilerParams(
            dimension_semantics=("parallel",)),
    )(chars2, char_table, cconv_w, cconv_b.reshape(1, D_CHAR))


def _tc_mix_call(wemb, cemb2, proj_w2, proj_b,
                 hw0_wt, hw0_bt, hw0_wg, hw0_bg, hw1_wt, hw1_bt, hw1_wg, hw1_bg):
    tok_spec = lambda w: pl.BlockSpec((TBM, w), lambda i: (i, 0))
    full = lambda *shape: pl.BlockSpec(shape, lambda i: (0,) * len(shape))
    return pl.pallas_call(
        _tc_mix_body,
        grid=(GRIDM,),
        in_specs=[
            tok_spec(D_WORD),
            tok_spec(D_CHAR),
            full(D_WORD + D_CHAR, D_MODEL),
            full(1, D_MODEL),
            full(D_MODEL, D_MODEL), full(1, D_MODEL),
            full(D_MODEL, D_MODEL), full(1, D_MODEL),
            full(D_MODEL, D_MODEL), full(1, D_MODEL),
            full(D_MODEL, D_MODEL), full(1, D_MODEL),
            pl.BlockSpec((LPB, 1, D_MODEL), lambda i: (i, 0, 0)),
        ],
        out_specs=tok_spec(D_MODEL),
        out_shape=jax.ShapeDtypeStruct((N_TOK, D_MODEL), jnp.float32),
        compiler_params=pltpu.CompilerParams(
            dimension_semantics=("parallel",)),
    )(wemb, cemb2, proj_w2, proj_b.reshape(1, D_MODEL),
      hw0_wt, hw0_bt.reshape(1, D_MODEL), hw0_wg, hw0_bg.reshape(1, D_MODEL),
      hw1_wt, hw1_bt.reshape(1, D_MODEL), hw1_wg, hw1_bg.reshape(1, D_MODEL),
      jnp.asarray(_POS).reshape(L, 1, D_MODEL))


def kernel(words, chars, word_table, char_table, cconv_w, cconv_b, proj_w,
           proj_b, hw0_wt, hw0_bt, hw0_wg, hw0_bg,
           hw1_wt, hw1_bt, hw1_wg, hw1_bg):
    idx3 = words.T.reshape(NW, NCH, CH)
    wemb = _sc_word_gather(word_table, idx3)
    chars2 = chars.transpose(2, 1, 0).reshape(WLEN, N_TOK)
    cemb2 = _tc_char_call(chars2, char_table, cconv_w, cconv_b)
    out2 = _tc_mix_call(wemb, cemb2,
                        proj_w.reshape(D_WORD + D_CHAR, D_MODEL), proj_b,
                        hw0_wt, hw0_bt, hw0_wg, hw0_bg,
                        hw1_wt, hw1_bt, hw1_wg, hw1_bg)
    return out2.reshape(L, B, D_MODEL).transpose(1, 0, 2)


# speedup vs baseline: 15.4851x; 1.0949x over previous
import functools

import numpy as np
import jax
import jax.numpy as jnp
from jax import lax
from jax.experimental import pallas as pl
from jax.experimental.pallas import tpu as pltpu
from jax.experimental.pallas import tpu_sc as plsc

B, L, WLEN = 1024, 50, 16
CV, D_CHAR, D_WORD, D_MODEL = 128, 64, 128, 128
N_TOK = B * L
W_VALID = WLEN - 4

NC, NS = 2, 16
NW = NC * NS
PER_W = N_TOK // NW
CH = 80
NCH = PER_W // CH


def _sc_word_gather(word_table, idx3):
    mesh = plsc.VectorSubcoreMesh(core_axis_name="c", subcore_axis_name="s")

    @functools.partial(
        pl.kernel,
        out_type=jax.ShapeDtypeStruct((N_TOK, D_WORD), jnp.float32),
        mesh=mesh,
        scratch_types=[
            pltpu.VMEM((NCH, CH), jnp.int32),
            pltpu.VMEM((CH, D_WORD), jnp.float32),
            pltpu.SemaphoreType.DMA,
        ],
    )
    def gather_kernel(table_hbm, idx_hbm, out_hbm, idx_v, rows_v, sem):
        wid = lax.axis_index("s") * NC + lax.axis_index("c")
        base = wid * PER_W
        pltpu.sync_copy(idx_hbm.at[wid], idx_v)

        def body(j, carry):
            pltpu.async_copy(table_hbm.at[idx_v.at[j]], rows_v, sem).wait()
            pltpu.sync_copy(rows_v, out_hbm.at[pl.ds(base + j * CH, CH)])
            return carry

        lax.fori_loop(0, NCH, body, 0)

    return gather_kernel(word_table, idx3)


TB = 2048
GRID = N_TOK // TB
TBM = 5120
GRIDM = N_TOK // TBM
LPB = TBM // B


def _pos_np():
    pos = np.arange(L)[:, None].astype(np.float64)
    i = np.arange(D_MODEL)[None, :].astype(np.float64)
    angle = pos / np.power(10000.0, (2.0 * (i // 2)) / D_MODEL)
    pe = np.where((np.arange(D_MODEL)[None, :] % 2) == 0,
                  np.sin(angle), np.cos(angle))
    return pe.astype(np.float32)


_POS = _pos_np()


def _tc_char_body(chars_ref, ctab_ref, cw_ref, cb_ref, cemb_ref):
    f32 = jnp.float32
    bf16 = jnp.bfloat16
    chars = chars_ref[...]
    cflat = chars.reshape(1, WLEN * TB)
    iota = lax.broadcasted_iota(jnp.int32, (CV, 1), 0)
    oht = (cflat == iota).astype(bf16)
    cw = cw_ref[...]
    ctab = ctab_ref[...]
    t_cat_t = jnp.concatenate(
        [lax.dot_general(cw[k], ctab, (((0,), (1,)), ((), ())),
                         preferred_element_type=f32)
         for k in range(5)], axis=0).astype(bf16)
    zt = jnp.dot(t_cat_t, oht, preferred_element_type=f32)
    yt = zt[0:D_CHAR, 0:W_VALID * TB]
    for k in range(1, 5):
        yt = yt + zt[k * D_CHAR:(k + 1) * D_CHAR, k * TB:(k + W_VALID) * TB]
    cembt = yt[:, 0:TB]
    for p in range(1, W_VALID):
        cembt = jnp.maximum(cembt, yt[:, p * TB:(p + 1) * TB])
    cemb_ref[...] = cembt.T + cb_ref[...]


def _tc_mix_body(wemb_ref, cemb_ref, pw_ref, pb_ref,
                 wt0_ref, bt0_ref, wg0_ref, bg0_ref,
                 wt1_ref, bt1_ref, wg1_ref, bg1_ref,
                 pos_ref, out_ref):
    f32 = jnp.float32
    bf16 = jnp.bfloat16
    wemb = wemb_ref[...]
    pw = pw_ref[...]
    h = (jnp.dot(wemb.astype(bf16), pw[0:D_WORD].astype(bf16),
                 preferred_element_type=f32)
         + jnp.dot(cemb_ref[...].astype(bf16), pw[D_WORD:].astype(bf16),
                   preferred_element_type=f32)
         + pb_ref[...])
    for wt_ref, bt_ref, wg_ref, bg_ref in (
            (wt0_ref, bt0_ref, wg0_ref, bg0_ref),
            (wt1_ref, bt1_ref, wg1_ref, bg1_ref)):
        hb = h.astype(bf16)
        gate = jnp.dot(hb, wg_ref[...].astype(bf16),
                       preferred_element_type=f32) + bg_ref[...]
        gate = 1.0 / (1.0 + jnp.exp(-gate))
        tr = jnp.dot(hb, wt_ref[...].astype(bf16),
                     preferred_element_type=f32) + bt_ref[...]
        tr = jnp.maximum(tr, 0.0)
        h = gate * h + (1.0 - gate) * tr
    pos = jnp.broadcast_to(pos_ref[:, 0, :][:, None, :],
                           (LPB, B, D_MODEL)).reshape(TBM, D_MODEL)
    out_ref[...] = h + pos


def _tc_char_call(chars2, char_table, cconv_w, cconv_b):
    full = lambda *shape: pl.BlockSpec(shape, lambda i: (0,) * len(shape))
    return pl.pallas_call(
        _tc_char_body,
        grid=(GRID,),
        in_specs=[
            pl.BlockSpec((WLEN, TB), lambda i: (0, i)),
            full(CV, D_CHAR),
            full(5, D_CHAR, D_CHAR),
            full(1, D_CHAR),
        ],
        out_specs=pl.BlockSpec((TB, D_CHAR), lambda i: (i, 0)),
        out_shape=jax.ShapeDtypeStruct((N_TOK, D_CHAR), jnp.float32),
        compiler_params=pltpu.CompilerParams(
            dimension_semantics=("parallel",)),
    )(chars2, char_table, cconv_w, cconv_b.reshape(1, D_CHAR))


def _tc_mix_call(wemb, cemb2, proj_w2, proj_b,
                 hw0_wt, hw0_bt, hw0_wg, hw0_bg, hw1_wt, hw1_bt, hw1_wg, hw1_bg):
    tok_spec = lambda w: pl.BlockSpec((TBM, w), lambda i: (i, 0))
    full = lambda *shape: pl.BlockSpec(shape, lambda i: (0,) * len(shape))
    return pl.pallas_call(
        _tc_mix_body,
        grid=(GRIDM,),
        in_specs=[
            tok_spec(D_WORD),
            tok_spec(D_CHAR),
            full(D_WORD + D_CHAR, D_MODEL),
            full(1, D_MODEL),
            full(D_MODEL, D_MODEL), full(1, D_MODEL),
            full(D_MODEL, D_MODEL), full(1, D_MODEL),
            full(D_MODEL, D_MODEL), full(1, D_MODEL),
            full(D_MODEL, D_MODEL), full(1, D_MODEL),
            pl.BlockSpec((LPB, 1, D_MODEL), lambda i: (i, 0, 0)),
        ],
        out_specs=tok_spec(D_MODEL),
        out_shape=jax.ShapeDtypeStruct((N_TOK, D_MODEL), jnp.float32),
        compiler_params=pltpu.CompilerParams(
            dimension_semantics=("parallel",)),
    )(wemb, cemb2, proj_w2, proj_b.reshape(1, D_MODEL),
      hw0_wt, hw0_bt.reshape(1, D_MODEL), hw0_wg, hw0_bg.reshape(1, D_MODEL),
      hw1_wt, hw1_bt.reshape(1, D_MODEL), hw1_wg, hw1_bg.reshape(1, D_MODEL),
      jnp.asarray(_POS).reshape(L, 1, D_MODEL))


def kernel(words, chars, word_table, char_table, cconv_w, cconv_b, proj_w,
           proj_b, hw0_wt, hw0_bt, hw0_wg, hw0_bg,
           hw1_wt, hw1_bt, hw1_wg, hw1_bg):
    idx3 = words.T.reshape(NW, NCH, CH)
    wemb = _sc_word_gather(word_table, idx3)
    chars2 = chars.transpose(2, 1, 0).reshape(WLEN, N_TOK)
    cemb2 = _tc_char_call(chars2, char_table, cconv_w, cconv_b)
    out2 = _tc_mix_call(wemb, cemb2,
                        proj_w.reshape(D_WORD + D_CHAR, D_MODEL), proj_b,
                        hw0_wt, hw0_bt, hw0_wg, hw0_bg,
                        hw1_wt, hw1_bt, hw1_wg, hw1_bg)
    return out2.reshape(L, B, D_MODEL).transpose(1, 0, 2)


# speedup vs baseline: 15.5559x; 1.0046x over previous
import functools

import numpy as np
import jax
import jax.numpy as jnp
from jax import lax
from jax.experimental import pallas as pl
from jax.experimental.pallas import tpu as pltpu
from jax.experimental.pallas import tpu_sc as plsc

B, L, WLEN = 1024, 50, 16
CV, D_CHAR, D_WORD, D_MODEL = 128, 64, 128, 128
N_TOK = B * L
W_VALID = WLEN - 4

NC, NS = 2, 16
NW = NC * NS
PER_W = N_TOK // NW
CH = 80
NCH = PER_W // CH


def _sc_word_gather(word_table, idx3):
    mesh = plsc.VectorSubcoreMesh(core_axis_name="c", subcore_axis_name="s")

    @functools.partial(
        pl.kernel,
        out_type=jax.ShapeDtypeStruct((N_TOK, D_WORD), jnp.float32),
        mesh=mesh,
        scratch_types=[
            pltpu.VMEM((NCH, CH), jnp.int32),
            pltpu.VMEM((CH, D_WORD), jnp.float32),
            pltpu.SemaphoreType.DMA,
        ],
    )
    def gather_kernel(table_hbm, idx_hbm, out_hbm, idx_v, rows_v, sem):
        wid = lax.axis_index("s") * NC + lax.axis_index("c")
        base = wid * PER_W
        pltpu.sync_copy(idx_hbm.at[wid], idx_v)

        def body(j, carry):
            pltpu.async_copy(table_hbm.at[idx_v.at[j]], rows_v, sem).wait()
            pltpu.sync_copy(rows_v, out_hbm.at[pl.ds(base + j * CH, CH)])
            return carry

        lax.fori_loop(0, NCH, body, 0)

    return gather_kernel(word_table, idx3)


TB = 2048
GRID = N_TOK // TB
TBM = 5120
GRIDM = N_TOK // TBM
LPB = TBM // B


def _pos_np():
    pos = np.arange(L)[:, None].astype(np.float64)
    i = np.arange(D_MODEL)[None, :].astype(np.float64)
    angle = pos / np.power(10000.0, (2.0 * (i // 2)) / D_MODEL)
    pe = np.where((np.arange(D_MODEL)[None, :] % 2) == 0,
                  np.sin(angle), np.cos(angle))
    return pe.astype(np.float32)


_POS = _pos_np()


def _tc_char_body(chars_ref, ctab_ref, cw_ref, cb_ref, cemb_ref):
    f32 = jnp.float32
    bf16 = jnp.bfloat16
    chars = chars_ref[...]
    cflat = chars.reshape(1, WLEN * TB)
    iota = lax.broadcasted_iota(jnp.int32, (CV, 1), 0)
    oht = (cflat == iota).astype(bf16)
    cw = cw_ref[...]
    ctab = ctab_ref[...]
    t_cat_t = jnp.concatenate(
        [lax.dot_general(cw[k], ctab, (((0,), (1,)), ((), ())),
                         preferred_element_type=f32)
         for k in range(5)], axis=0).astype(bf16)
    zt = jnp.dot(t_cat_t, oht, preferred_element_type=f32)
    yt = zt[0:D_CHAR, 0:W_VALID * TB]
    for k in range(1, 5):
        yt = yt + zt[k * D_CHAR:(k + 1) * D_CHAR, k * TB:(k + W_VALID) * TB]
    cembt = yt[:, 0:TB]
    for p in range(1, W_VALID):
        cembt = jnp.maximum(cembt, yt[:, p * TB:(p + 1) * TB])
    cemb_ref[...] = (cembt.T + cb_ref[...]).astype(bf16)


def _tc_mix_body(wemb_ref, cemb_ref, pw_ref, pb_ref,
                 wt0_ref, bt0_ref, wg0_ref, bg0_ref,
                 wt1_ref, bt1_ref, wg1_ref, bg1_ref,
                 pos_ref, out_ref):
    f32 = jnp.float32
    bf16 = jnp.bfloat16
    wemb = wemb_ref[...]
    pw = pw_ref[...]
    h = (jnp.dot(wemb.astype(bf16), pw[0:D_WORD].astype(bf16),
                 preferred_element_type=f32)
         + jnp.dot(cemb_ref[...].astype(bf16), pw[D_WORD:].astype(bf16),
                   preferred_element_type=f32)
         + pb_ref[...])
    for wt_ref, bt_ref, wg_ref, bg_ref in (
            (wt0_ref, bt0_ref, wg0_ref, bg0_ref),
            (wt1_ref, bt1_ref, wg1_ref, bg1_ref)):
        hb = h.astype(bf16)
        gate = jnp.dot(hb, wg_ref[...].astype(bf16),
                       preferred_element_type=f32) + bg_ref[...]
        gate = 1.0 / (1.0 + jnp.exp(-gate))
        tr = jnp.dot(hb, wt_ref[...].astype(bf16),
                     preferred_element_type=f32) + bt_ref[...]
        tr = jnp.maximum(tr, 0.0)
        h = gate * h + (1.0 - gate) * tr
    pos = jnp.broadcast_to(pos_ref[:, 0, :][:, None, :],
                           (LPB, B, D_MODEL)).reshape(TBM, D_MODEL)
    out_ref[...] = h + pos


def _tc_char_call(chars2, char_table, cconv_w, cconv_b):
    full = lambda *shape: pl.BlockSpec(shape, lambda i: (0,) * len(shape))
    return pl.pallas_call(
        _tc_char_body,
        grid=(GRID,),
        in_specs=[
            pl.BlockSpec((WLEN, TB), lambda i: (0, i)),
            full(CV, D_CHAR),
            full(5, D_CHAR, D_CHAR),
            full(1, D_CHAR),
        ],
        out_specs=pl.BlockSpec((TB, D_CHAR), lambda i: (i, 0)),
        out_shape=jax.ShapeDtypeStruct((N_TOK, D_CHAR), jnp.bfloat16),
        compiler_params=pltpu.CompilerParams(
            dimension_semantics=("parallel",)),
    )(chars2, char_table, cconv_w, cconv_b.reshape(1, D_CHAR))


def _tc_mix_call(wemb, cemb2, proj_w2, proj_b,
                 hw0_wt, hw0_bt, hw0_wg, hw0_bg, hw1_wt, hw1_bt, hw1_wg, hw1_bg):
    tok_spec = lambda w: pl.BlockSpec((TBM, w), lambda i: (i, 0))
    full = lambda *shape: pl.BlockSpec(shape, lambda i: (0,) * len(shape))
    return pl.pallas_call(
        _tc_mix_body,
        grid=(GRIDM,),
        in_specs=[
            tok_spec(D_WORD),
            tok_spec(D_CHAR),
            full(D_WORD + D_CHAR, D_MODEL),
            full(1, D_MODEL),
            full(D_MODEL, D_MODEL), full(1, D_MODEL),
            full(D_MODEL, D_MODEL), full(1, D_MODEL),
            full(D_MODEL, D_MODEL), full(1, D_MODEL),
            full(D_MODEL, D_MODEL), full(1, D_MODEL),
            pl.BlockSpec((LPB, 1, D_MODEL), lambda i: (i, 0, 0)),
        ],
        out_specs=tok_spec(D_MODEL),
        out_shape=jax.ShapeDtypeStruct((N_TOK, D_MODEL), jnp.float32),
        compiler_params=pltpu.CompilerParams(
            dimension_semantics=("parallel",)),
    )(wemb, cemb2, proj_w2, proj_b.reshape(1, D_MODEL),
      hw0_wt, hw0_bt.reshape(1, D_MODEL), hw0_wg, hw0_bg.reshape(1, D_MODEL),
      hw1_wt, hw1_bt.reshape(1, D_MODEL), hw1_wg, hw1_bg.reshape(1, D_MODEL),
      jnp.asarray(_POS).reshape(L, 1, D_MODEL))


def kernel(words, chars, word_table, char_table, cconv_w, cconv_b, proj_w,
           proj_b, hw0_wt, hw0_bt, hw0_wg, hw0_bg,
           hw1_wt, hw1_bt, hw1_wg, hw1_bg):
    idx3 = words.T.reshape(NW, NCH, CH)
    wemb = _sc_word_gather(word_table, idx3)
    chars2 = chars.transpose(2, 1, 0).reshape(WLEN, N_TOK)
    cemb2 = _tc_char_call(chars2, char_table, cconv_w, cconv_b)
    out2 = _tc_mix_call(wemb, cemb2,
                        proj_w.reshape(D_WORD + D_CHAR, D_MODEL), proj_b,
                        hw0_wt, hw0_bt, hw0_wg, hw0_bg,
                        hw1_wt, hw1_bt, hw1_wg, hw1_bg)
    return out2.reshape(L, B, D_MODEL).transpose(1, 0, 2)
